# baseline (speedup 1.0000x reference)
import functools

import jax
import jax.numpy as jnp
from jax import lax
from jax.experimental import pallas as pl
from jax.experimental.pallas import tpu as pltpu
from jax.experimental.pallas import tpu_sc as plsc

EMBED = 256


@functools.lru_cache(maxsize=None)
def _make_lookup(n_rows: int, vocab: int):
    info = plsc.get_sparse_core_info()
    nw = info.num_cores * info.num_subcores
    assert n_rows % (8 * nw) == 0
    per_w = n_rows // nw
    mesh = plsc.VectorSubcoreMesh(core_axis_name="c", subcore_axis_name="s")

    @functools.partial(
        pl.kernel,
        mesh=mesh,
        out_type=jax.ShapeDtypeStruct((n_rows, EMBED), jnp.float32),
        scratch_types=[
            pltpu.VMEM((vocab, EMBED), jnp.float32),
            pltpu.VMEM((per_w,), jnp.int32),
            pltpu.VMEM((128, EMBED), jnp.float32),
            pltpu.SemaphoreType.DMA,
            pltpu.SemaphoreType.DMA,
        ],
    )
    def lookup(table_hbm, idx_hbm, out_hbm, table_v, idx_v, drain_v, sem,
               lsem):
        wid = lax.axis_index("s") * info.num_cores + lax.axis_index("c")
        base = wid * per_w
        t_copy = pltpu.async_copy(table_hbm, table_v, lsem)
        i_copy = pltpu.async_copy(idx_hbm.at[pl.ds(base, per_w)], idx_v, lsem)
        t_copy.wait()
        i_copy.wait()

        def group_body(g, carry):
            iv = idx_v[pl.ds(g * 16, 16)]
            for r in range(16):
                s = iv[r]
                pltpu.async_copy(
                    table_v.at[s], out_hbm.at[base + g * 16 + r], sem
                )
            return carry

        lax.fori_loop(0, per_w // 16, group_body, 0)

        for _ in range(per_w // 128):
            pltpu.make_async_copy(
                out_hbm.at[pl.ds(0, 128)], drain_v, sem
            ).wait()

    return lookup


def kernel(x, table):
    b, s = x.shape
    n = b * s
    idx = x.reshape(n).astype(jnp.int32)
    out = _make_lookup(n, table.shape[0])(table.astype(jnp.float32), idx)
    return out.reshape(b, s, EMBED)


# speedup vs baseline: 1.0554x; 1.0554x over previous
import functools

import jax
import jax.numpy as jnp
from jax import lax
from jax.experimental import pallas as pl
from jax.experimental.pallas import tpu as pltpu
from jax.experimental.pallas import tpu_sc as plsc

EMBED = 256


@functools.lru_cache(maxsize=None)
def _make_lookup(n_rows: int, vocab: int):
    info = plsc.get_sparse_core_info()
    nw = info.num_cores * info.num_subcores
    assert n_rows % (8 * nw) == 0
    per_w = n_rows // nw
    mesh = plsc.VectorSubcoreMesh(core_axis_name="c", subcore_axis_name="s")

    @functools.partial(
        pl.kernel,
        mesh=mesh,
        out_type=jax.ShapeDtypeStruct((n_rows, EMBED), jnp.float32),
        scratch_types=[
            pltpu.VMEM((vocab, EMBED), jnp.float32),
            pltpu.VMEM((per_w,), jnp.int32),
            pltpu.VMEM((128, EMBED), jnp.float32),
            pltpu.SemaphoreType.DMA,
        ],
    )
    def lookup(table_hbm, idx_hbm, out_hbm, table_v, idx_v, drain_v, sem):
        wid = lax.axis_index("s") * info.num_cores + lax.axis_index("c")
        base = wid * per_w
        pltpu.sync_copy(table_hbm, table_v)
        pltpu.sync_copy(idx_hbm.at[pl.ds(base, per_w)], idx_v)

        def group_body(g, carry):
            iv = idx_v[pl.ds(g * 16, 16)]
            for r in range(16):
                s = iv[r]
                pltpu.async_copy(
                    table_v.at[s], out_hbm.at[base + g * 16 + r], sem
                )
            return carry

        lax.fori_loop(0, per_w // 16, group_body, 0)

        for _ in range(per_w // 128):
            pltpu.make_async_copy(
                out_hbm.at[pl.ds(0, 128)], drain_v, sem
            ).wait()

    return lookup


def kernel(x, table):
    b, s = x.shape
    n = b * s
    idx = x.reshape(n).astype(jnp.int32)
    out = _make_lookup(n, table.shape[0])(table.astype(jnp.float32), idx)
    return out.reshape(b, s, EMBED)
